# Initial kernel scaffold; baseline (speedup 1.0000x reference)
import jax
import jax.numpy as jnp
from jax.experimental import pallas as pl


def kernel(states, lut):
    raise NotImplementedError("write your pallas kernel here")


# speedup vs baseline: 7.3773x; 7.3773x over previous
import functools

import jax
import jax.numpy as jnp
from jax import lax
from jax.experimental import pallas as pl
from jax.experimental.pallas import tpu as pltpu
from jax.experimental.pallas import tpu_sc as plsc

CHUNK = 16
NSTATES = 65536
NC, NS, L = 2, 16, 16
NW = NC * NS
BLK = 8192


def kernel(states, lut):
    n = states.size
    per_w = n // 2
    nblk = per_w // BLK
    states_flat = states.reshape(-1)

    mesh = plsc.VectorSubcoreMesh(core_axis_name="c", subcore_axis_name="s")

    @functools.partial(
        pl.kernel,
        out_type=jax.ShapeDtypeStruct((CHUNK, n), jnp.float32),
        mesh=mesh,
        scratch_types=[
            pltpu.VMEM((NSTATES,), jnp.float32),
            pltpu.VMEM((BLK,), jnp.int32),
            pltpu.VMEM((BLK,), jnp.float32),
        ],
        compiler_params=pltpu.CompilerParams(needs_layout_passes=False),
    )
    def k(states_hbm, lut_hbm, out_hbm, lut_v, idx_v, res_v):
        wid = lax.axis_index("s") * NC + lax.axis_index("c")
        row = wid // 2
        half = wid % 2
        base = half * per_w

        pltpu.sync_copy(lut_hbm.at[row], lut_v)

        def blk_body(b, carry):
            off = base + b * BLK
            pltpu.sync_copy(states_hbm.at[pl.ds(off, BLK)], idx_v)

            def g_body(g, c2):
                iv = idx_v[pl.ds(g * L, L)]
                res_v[pl.ds(g * L, L)] = plsc.load_gather(lut_v, [iv])
                return c2

            lax.fori_loop(0, BLK // L, g_body, 0, unroll=8)
            pltpu.sync_copy(res_v, out_hbm.at[row, pl.ds(off, BLK)])
            return carry

        lax.fori_loop(0, nblk, blk_body, 0)

    out = k(states_flat, lut)
    return out.reshape(CHUNK, *states.shape)


# speedup vs baseline: 8.2632x; 1.1201x over previous
import functools

import jax
import jax.numpy as jnp
from jax import lax
from jax.experimental import pallas as pl
from jax.experimental.pallas import tpu as pltpu
from jax.experimental.pallas import tpu_sc as plsc

CHUNK = 16
NSTATES = 65536
NC, NS, L = 2, 16, 16
NW = NC * NS
BLK = 8192
NBUF = 2


def kernel(states, lut):
    n = states.size
    per_w = n // 2
    nblk = per_w // BLK
    states_flat = states.reshape(-1)

    mesh = plsc.VectorSubcoreMesh(core_axis_name="c", subcore_axis_name="s")

    @functools.partial(
        pl.kernel,
        out_type=jax.ShapeDtypeStruct((CHUNK, n), jnp.float32),
        mesh=mesh,
        scratch_types=[
            pltpu.VMEM((NSTATES,), jnp.float32),
            pltpu.VMEM((NBUF, BLK), jnp.int32),
            pltpu.VMEM((NBUF, BLK), jnp.float32),
            pltpu.SemaphoreType.DMA,
            [pltpu.SemaphoreType.DMA] * NBUF,
            [pltpu.SemaphoreType.DMA] * NBUF,
        ],
        compiler_params=pltpu.CompilerParams(needs_layout_passes=False),
    )
    def k(states_hbm, lut_hbm, out_hbm, lut_v, idx_v, res_v, lut_sem,
          in_sems, out_sems):
        wid = lax.axis_index("s") * NC + lax.axis_index("c")
        row = wid // 2
        half = wid % 2
        base = half * per_w

        lut_cp = pltpu.async_copy(lut_hbm.at[row], lut_v, lut_sem)

        def in_cp(b, j):
            return pltpu.make_async_copy(
                states_hbm.at[pl.ds(base + b * BLK, BLK)], idx_v.at[j],
                in_sems[j])

        def out_cp(b, j):
            return pltpu.make_async_copy(
                res_v.at[j], out_hbm.at[row, pl.ds(base + b * BLK, BLK)],
                out_sems[j])

        for j in range(NBUF):
            in_cp(j, j).start()
        lut_cp.wait()

        def blk_body(i, carry):
            for j in range(NBUF):
                b = i * NBUF + j
                in_cp(b, j).wait()
                pl.when(b >= NBUF)(lambda: out_cp(b - NBUF, j).wait())

                def g_body(g, c2):
                    iv = idx_v[j, pl.ds(g * L, L)]
                    res_v[j, pl.ds(g * L, L)] = plsc.load_gather(lut_v, [iv])
                    return c2

                lax.fori_loop(0, BLK // L, g_body, 0, unroll=8)
                out_cp(b, j).start()
                pl.when(b + NBUF < nblk)(lambda: in_cp(b + NBUF, j).start())
            return carry

        lax.fori_loop(0, nblk // NBUF, blk_body, 0)
        for j in range(NBUF):
            out_cp(nblk - NBUF + j, j).wait()

    out = k(states_flat, lut)
    return out.reshape(CHUNK, *states.shape)


# speedup vs baseline: 17.5214x; 2.1204x over previous
import functools

import jax
import jax.numpy as jnp
from jax import lax
from jax.experimental import pallas as pl
from jax.experimental.pallas import tpu as pltpu
from jax.experimental.pallas import tpu_sc as plsc

CHUNK = 16
NSTATES = 65536
NC, NS, L = 2, 16, 16
NW = NC * NS
BLK = 8192
NBUF = 2


def kernel(states, lut):
    n = states.size
    per_w = n // 2
    nblk = per_w // BLK
    states_flat = states.reshape(-1)

    mesh = plsc.VectorSubcoreMesh(core_axis_name="c", subcore_axis_name="s")

    @functools.partial(
        pl.kernel,
        out_type=jax.ShapeDtypeStruct((CHUNK, n), jnp.float32),
        mesh=mesh,
        scratch_types=[
            pltpu.VMEM((NSTATES,), jnp.float32),
            pltpu.VMEM((NBUF, BLK), jnp.int32),
            pltpu.VMEM((NBUF, BLK), jnp.float32),
            pltpu.SemaphoreType.DMA,
            [pltpu.SemaphoreType.DMA] * NBUF,
            [pltpu.SemaphoreType.DMA] * NBUF,
        ],
        compiler_params=pltpu.CompilerParams(needs_layout_passes=False),
    )
    def k(states_hbm, lut_hbm, out_hbm, lut_v, idx_v, res_v, lut_sem,
          in_sems, out_sems):
        wid = lax.axis_index("s") * NC + lax.axis_index("c")
        row = wid // 2
        half = wid % 2
        base = half * per_w

        lut_cp = pltpu.async_copy(lut_hbm.at[row], lut_v, lut_sem)

        def in_cp(b, j):
            return pltpu.make_async_copy(
                states_hbm.at[pl.ds(base + b * BLK, BLK)], idx_v.at[j],
                in_sems[j])

        def out_cp(b, j):
            return pltpu.make_async_copy(
                res_v.at[j], out_hbm.at[row, pl.ds(base + b * BLK, BLK)],
                out_sems[j])

        for j in range(NBUF):
            in_cp(j, j).start()
        lut_cp.wait()

        def blk_body(i, carry):
            for j in range(NBUF):
                b = i * NBUF + j
                in_cp(b, j).wait()
                pl.when(b >= NBUF)(lambda: out_cp(b - NBUF, j).wait())

                @plsc.parallel_loop(0, BLK, step=L, unroll=8)
                def g_body(g):
                    iv = idx_v[j, pl.ds(g, L)]
                    res_v[j, pl.ds(g, L)] = plsc.load_gather(lut_v, [iv])
                out_cp(b, j).start()
                pl.when(b + NBUF < nblk)(lambda: in_cp(b + NBUF, j).start())
            return carry

        lax.fori_loop(0, nblk // NBUF, blk_body, 0)
        for j in range(NBUF):
            out_cp(nblk - NBUF + j, j).wait()

    out = k(states_flat, lut)
    return out.reshape(CHUNK, *states.shape)


# speedup vs baseline: 17.5482x; 1.0015x over previous
import functools

import jax
import jax.numpy as jnp
from jax import lax
from jax.experimental import pallas as pl
from jax.experimental.pallas import tpu as pltpu
from jax.experimental.pallas import tpu_sc as plsc

CHUNK = 16
NSTATES = 65536
NC, NS, L = 2, 16, 16
NW = NC * NS
BLK = 8192
NBUF = 2


def kernel(states, lut):
    n = states.size
    per_w = n // 2
    nblk = per_w // BLK
    states_flat = states.reshape(-1)

    mesh = plsc.VectorSubcoreMesh(core_axis_name="c", subcore_axis_name="s")

    @functools.partial(
        pl.kernel,
        out_type=jax.ShapeDtypeStruct((CHUNK, n), jnp.float32),
        mesh=mesh,
        scratch_types=[
            pltpu.VMEM((NSTATES,), jnp.float32),
            pltpu.VMEM((NBUF, BLK), jnp.int32),
            pltpu.VMEM((NBUF, BLK), jnp.float32),
            pltpu.SemaphoreType.DMA,
            [pltpu.SemaphoreType.DMA] * NBUF,
            [pltpu.SemaphoreType.DMA] * NBUF,
        ],
        compiler_params=pltpu.CompilerParams(needs_layout_passes=False),
    )
    def k(states_hbm, lut_hbm, out_hbm, lut_v, idx_v, res_v, lut_sem,
          in_sems, out_sems):
        wid = lax.axis_index("s") * NC + lax.axis_index("c")
        row = wid // 2
        half = wid % 2
        base = half * per_w

        lut_cp = pltpu.async_copy(lut_hbm.at[row], lut_v, lut_sem)

        def in_cp(b, j):
            return pltpu.make_async_copy(
                states_hbm.at[pl.ds(base + b * BLK, BLK)], idx_v.at[j],
                in_sems[j])

        def out_cp(b, j):
            return pltpu.make_async_copy(
                res_v.at[j], out_hbm.at[row, pl.ds(base + b * BLK, BLK)],
                out_sems[j])

        for j in range(NBUF):
            in_cp(j, j).start()
        lut_cp.wait()

        def blk_body(i, carry):
            for j in range(NBUF):
                b = i * NBUF + j
                in_cp(b, j).wait()
                pl.when(b >= NBUF)(lambda: out_cp(b - NBUF, j).wait())

                @plsc.parallel_loop(0, BLK, step=L, unroll=16)
                def g_body(g):
                    iv = idx_v[j, pl.ds(g, L)]
                    res_v[j, pl.ds(g, L)] = plsc.load_gather(lut_v, [iv])
                out_cp(b, j).start()
                pl.when(b + NBUF < nblk)(lambda: in_cp(b + NBUF, j).start())
            return carry

        lax.fori_loop(0, nblk // NBUF, blk_body, 0)
        for j in range(NBUF):
            out_cp(nblk - NBUF + j, j).wait()

    out = k(states_flat, lut)
    return out.reshape(CHUNK, *states.shape)
